# Initial kernel scaffold; baseline (speedup 1.0000x reference)
import functools

import jax
import jax.numpy as jnp
from jax import lax
from jax.experimental import pallas as pl
from jax.experimental.pallas import tpu as pltpu
from jax.experimental.pallas import tpu_sc as plsc

N = 10000
E = 320000
N_PAD = 10240
N_TRASH = 10000
NC, NS = 2, 16
NW = NC * NS
K = 128
CHUNKS = 79
E_PAD = NW * K * CHUNKS
ROWS_PT = N_PAD // NS


def _elu(a):
    return jnp.where(a > 0, a, jnp.exp(jnp.minimum(a, 0.0)) - 1.0)


def _deg_kernel(dst_hbm, out_hbm, dst_v, ones_v, zbuf_v, acc_s):
    cid = lax.axis_index("c")
    sid = lax.axis_index("s")
    zero16 = jnp.zeros((16,), jnp.float32)
    one16 = jnp.ones((16,), jnp.float32)
    for i in range(K // 16):
        ones_v[pl.ds(i * 16, 16)] = one16

    def zb(i, c):
        zbuf_v[pl.ds(i * 16, 16)] = zero16
        return c
    lax.fori_loop(0, ROWS_PT // 16, zb, 0)
    pltpu.sync_copy(zbuf_v, acc_s.at[pl.ds(sid * ROWS_PT, ROWS_PT)])
    plsc.subcore_barrier()

    def body(ci, c):
        base = (cid * NS + sid) * (K * CHUNKS) + ci * K
        pltpu.sync_copy(dst_hbm.at[pl.ds(base, K)], dst_v)
        pltpu.sync_copy(ones_v, acc_s.at[dst_v], add=True)
        return c
    lax.fori_loop(0, CHUNKS, body, 0)
    plsc.subcore_barrier()
    pltpu.sync_copy(acc_s.at[pl.ds(sid * ROWS_PT, ROWS_PT)], zbuf_v)
    pltpu.sync_copy(zbuf_v, out_hbm.at[cid, pl.ds(sid * ROWS_PT, ROWS_PT)])


_deg_pass = functools.partial(
    pl.kernel,
    out_type=jax.ShapeDtypeStruct((NC, N_PAD), jnp.float32),
    mesh=plsc.VectorSubcoreMesh(core_axis_name="c", subcore_axis_name="s"),
    scratch_types=[
        pltpu.VMEM((K,), jnp.int32),
        pltpu.VMEM((K,), jnp.float32),
        pltpu.VMEM((ROWS_PT,), jnp.float32),
        pltpu.VMEM_SHARED((N_PAD,), jnp.float32),
    ],
)(_deg_kernel)


def _make_agg(D):
    zsteps = ROWS_PT // K

    def body(y_hbm, src_hbm, dst_hbm, out_hbm, src_v, dst_v, rows_v, acc_s,
             sem):
        cid = lax.axis_index("c")
        sid = lax.axis_index("s")
        zero16 = jnp.zeros((16,), jnp.float32)

        def zb(i, c):
            for j in range(D // 16):
                rows_v[i, pl.ds(j * 16, 16)] = zero16
            return c
        lax.fori_loop(0, K, zb, 0)
        for j in range(zsteps):
            pltpu.sync_copy(rows_v,
                            acc_s.at[pl.ds(sid * ROWS_PT + j * K, K)])
        plsc.subcore_barrier()

        def chunk(ci, c):
            base = (cid * NS + sid) * (K * CHUNKS) + ci * K
            pltpu.sync_copy(src_hbm.at[pl.ds(base, K)], src_v)
            pltpu.sync_copy(dst_hbm.at[pl.ds(base, K)], dst_v)
            pltpu.async_copy(y_hbm.at[src_v], rows_v, sem).wait()
            pltpu.sync_copy(rows_v, acc_s.at[dst_v], add=True)
            return c
        lax.fori_loop(0, CHUNKS, chunk, 0)
        plsc.subcore_barrier()
        for j in range(zsteps):
            sl = pl.ds(sid * ROWS_PT + j * K, K)
            pltpu.sync_copy(acc_s.at[sl], rows_v)
            pltpu.sync_copy(rows_v, out_hbm.at[cid, sl])

    return functools.partial(
        pl.kernel,
        out_type=jax.ShapeDtypeStruct((NC, N_PAD, D), jnp.float32),
        mesh=plsc.VectorSubcoreMesh(core_axis_name="c", subcore_axis_name="s"),
        scratch_types=[
            pltpu.VMEM((K,), jnp.int32),
            pltpu.VMEM((K,), jnp.int32),
            pltpu.VMEM((K, D), jnp.float32),
            pltpu.VMEM_SHARED((N_PAD, D), jnp.float32),
            pltpu.SemaphoreType.DMA,
        ],
    )(body)


_agg64 = _make_agg(64)
_agg32 = _make_agg(32)


def _dis_from(degp_ref):
    deg = degp_ref[0, :N, :] + degp_ref[1, :N, :] + 1.0
    return lax.rsqrt(deg)


def _tc1_body(x_ref, w1_ref, degp_ref, y1_ref):
    dis = _dis_from(degp_ref)
    y1_ref[...] = jnp.dot(x_ref[...], w1_ref[...],
                          preferred_element_type=jnp.float32) * dis


def _tc2_body(y1_ref, s1_ref, degp_ref, b1_ref, w2_ref, y2_ref):
    dis = _dis_from(degp_ref)
    s = s1_ref[0, :N, :] + s1_ref[1, :N, :]
    agg = dis * (y1_ref[...] + s) + jnp.reshape(b1_ref[...], (1, -1))
    h1 = _elu(agg)
    y2_ref[...] = jnp.dot(h1, w2_ref[...],
                          preferred_element_type=jnp.float32) * dis


def _tc3_body(y2_ref, s2_ref, degp_ref, b2_ref, wc_ref, bc_ref, out_ref):
    dis = _dis_from(degp_ref)
    s = s2_ref[0, :N, :] + s2_ref[1, :N, :]
    agg = dis * (y2_ref[...] + s) + jnp.reshape(b2_ref[...], (1, -1))
    h2 = _elu(agg)
    out_ref[...] = jnp.dot(h2, wc_ref[...],
                           preferred_element_type=jnp.float32) + bc_ref[...]


def _tc1(x, W1, degp):
    return pl.pallas_call(
        _tc1_body,
        out_shape=jax.ShapeDtypeStruct((N, 64), jnp.float32),
    )(x, W1, degp)


def _tc2(y1, s1, degp, b1, W2):
    return pl.pallas_call(
        _tc2_body,
        out_shape=jax.ShapeDtypeStruct((N, 32), jnp.float32),
    )(y1, s1, degp, b1, W2)


def _tc3(y2, s2, degp, b2, Wc, bc):
    return pl.pallas_call(
        _tc3_body,
        out_shape=jax.ShapeDtypeStruct((N, 1), jnp.float32),
    )(y2, s2, degp, b2, Wc, bc)


def kernel(x, edge_index, W1, b1, W2, b2, Wc, bc):
    pad = E_PAD - E
    src = jnp.concatenate([edge_index[0],
                           jnp.zeros((pad,), jnp.int32)])
    dst = jnp.concatenate([edge_index[1],
                           jnp.full((pad,), N_TRASH, jnp.int32)])

    degp = _deg_pass(dst)
    degp3 = jnp.reshape(degp, (NC, N_PAD, 1))

    y1 = _tc1(x, W1, degp3)
    s1 = _agg64(y1, src, dst)
    y2 = _tc2(y1, s1, degp3, b1, W2)
    s2 = _agg32(y2, src, dst)
    out = _tc3(y2, s2, degp3, b2, Wc, bc)
    return out[:, 0]


# speedup vs baseline: 16.5364x; 16.5364x over previous
import functools

import jax
import jax.numpy as jnp
from jax import lax
from jax.experimental import pallas as pl
from jax.experimental.pallas import tpu as pltpu
from jax.experimental.pallas import tpu_sc as plsc

N = 10000
E = 320000
N_PAD = 10240
N_TRASH = 10000
NC, NS = 2, 16
NW = NC * NS
K = 128
CHUNKS = 79
E_PAD = NW * K * CHUNKS
ROWS_PT = N_PAD // NS


def _elu(a):
    return jnp.where(a > 0, a, jnp.exp(jnp.minimum(a, 0.0)) - 1.0)


def _deg_kernel(dst_hbm, out_hbm, dst_v, ones_v, zbuf_v, acc_s):
    cid = lax.axis_index("c")
    sid = lax.axis_index("s")
    zero16 = jnp.zeros((16,), jnp.float32)
    one16 = jnp.ones((16,), jnp.float32)
    for i in range(K // 16):
        ones_v[pl.ds(i * 16, 16)] = one16

    def zb(i, c):
        zbuf_v[pl.ds(i * 16, 16)] = zero16
        return c
    lax.fori_loop(0, ROWS_PT // 16, zb, 0)
    pltpu.sync_copy(zbuf_v, acc_s.at[pl.ds(sid * ROWS_PT, ROWS_PT)])
    plsc.subcore_barrier()

    def body(ci, c):
        base = (cid * NS + sid) * (K * CHUNKS) + ci * K
        pltpu.sync_copy(dst_hbm.at[pl.ds(base, K)], dst_v)
        pltpu.sync_copy(ones_v, acc_s.at[dst_v], add=True)
        return c
    lax.fori_loop(0, CHUNKS, body, 0)
    plsc.subcore_barrier()
    pltpu.sync_copy(acc_s.at[pl.ds(sid * ROWS_PT, ROWS_PT)], zbuf_v)
    pltpu.sync_copy(zbuf_v, out_hbm.at[cid, pl.ds(sid * ROWS_PT, ROWS_PT)])


_deg_pass = functools.partial(
    pl.kernel,
    out_type=jax.ShapeDtypeStruct((NC, N_PAD), jnp.float32),
    mesh=plsc.VectorSubcoreMesh(core_axis_name="c", subcore_axis_name="s"),
    scratch_types=[
        pltpu.VMEM((K,), jnp.int32),
        pltpu.VMEM((K,), jnp.float32),
        pltpu.VMEM((ROWS_PT,), jnp.float32),
        pltpu.VMEM_SHARED((N_PAD,), jnp.float32),
    ],
)(_deg_kernel)


def _make_agg(D):
    zsteps = ROWS_PT // K

    def body(y_hbm, src_hbm, dst_hbm, out_hbm, src_v, dst_v, rows_v, acc_s,
             sem):
        cid = lax.axis_index("c")
        sid = lax.axis_index("s")
        zero16 = jnp.zeros((16,), jnp.float32)

        def zb(i, c):
            for j in range(D // 16):
                rows_v[i, pl.ds(j * 16, 16)] = zero16
            return c
        lax.fori_loop(0, K, zb, 0)
        for j in range(zsteps):
            pltpu.sync_copy(rows_v,
                            acc_s.at[pl.ds(sid * ROWS_PT + j * K, K)])
        plsc.subcore_barrier()

        def chunk(ci, c):
            base = (cid * NS + sid) * (K * CHUNKS) + ci * K
            pltpu.sync_copy(src_hbm.at[pl.ds(base, K)], src_v)
            pltpu.sync_copy(dst_hbm.at[pl.ds(base, K)], dst_v)
            pltpu.async_copy(y_hbm.at[src_v], rows_v, sem).wait()
            pltpu.sync_copy(rows_v, acc_s.at[dst_v], add=True)
            return c
        lax.fori_loop(0, CHUNKS, chunk, 0)
        plsc.subcore_barrier()
        for j in range(zsteps):
            sl = pl.ds(sid * ROWS_PT + j * K, K)
            pltpu.sync_copy(acc_s.at[sl], rows_v)
            pltpu.sync_copy(rows_v, out_hbm.at[cid, sl])

    return functools.partial(
        pl.kernel,
        out_type=jax.ShapeDtypeStruct((NC, N_PAD, D), jnp.float32),
        mesh=plsc.VectorSubcoreMesh(core_axis_name="c", subcore_axis_name="s"),
        scratch_types=[
            pltpu.VMEM((K,), jnp.int32),
            pltpu.VMEM((K,), jnp.int32),
            pltpu.VMEM((K, D), jnp.float32),
            pltpu.VMEM_SHARED((N_PAD, D), jnp.float32),
            pltpu.SemaphoreType.DMA,
        ],
        compiler_params=pltpu.CompilerParams(use_tc_tiling_on_sc=False),
    )(body)


_agg64 = _make_agg(64)
_agg32 = _make_agg(32)


def _dis_from(degp_ref):
    deg = degp_ref[0, :N, :] + degp_ref[1, :N, :] + 1.0
    return lax.rsqrt(deg)


def _tc1_body(x_ref, w1_ref, degp_ref, y1_ref):
    dis = _dis_from(degp_ref)
    y1_ref[...] = jnp.dot(x_ref[...], w1_ref[...],
                          preferred_element_type=jnp.float32) * dis


def _tc2_body(y1_ref, s1_ref, degp_ref, b1_ref, w2_ref, y2_ref):
    dis = _dis_from(degp_ref)
    s = s1_ref[0, :N, :] + s1_ref[1, :N, :]
    agg = dis * (y1_ref[...] + s) + jnp.reshape(b1_ref[...], (1, -1))
    h1 = _elu(agg)
    y2_ref[...] = jnp.dot(h1, w2_ref[...],
                          preferred_element_type=jnp.float32) * dis


def _tc3_body(y2_ref, s2_ref, degp_ref, b2_ref, wc_ref, bc_ref, out_ref):
    dis = _dis_from(degp_ref)
    s = s2_ref[0, :N, :] + s2_ref[1, :N, :]
    agg = dis * (y2_ref[...] + s) + jnp.reshape(b2_ref[...], (1, -1))
    h2 = _elu(agg)
    out_ref[...] = jnp.dot(h2, wc_ref[...],
                           preferred_element_type=jnp.float32) + bc_ref[...]


def _tc1(x, W1, degp):
    return pl.pallas_call(
        _tc1_body,
        out_shape=jax.ShapeDtypeStruct((N, 64), jnp.float32),
    )(x, W1, degp)


def _tc2(y1, s1, degp, b1, W2):
    return pl.pallas_call(
        _tc2_body,
        out_shape=jax.ShapeDtypeStruct((N, 32), jnp.float32),
    )(y1, s1, degp, b1, W2)


def _tc3(y2, s2, degp, b2, Wc, bc):
    return pl.pallas_call(
        _tc3_body,
        out_shape=jax.ShapeDtypeStruct((N, 1), jnp.float32),
    )(y2, s2, degp, b2, Wc, bc)


def kernel(x, edge_index, W1, b1, W2, b2, Wc, bc):
    pad = E_PAD - E
    src = jnp.concatenate([edge_index[0],
                           jnp.zeros((pad,), jnp.int32)])
    dst = jnp.concatenate([edge_index[1],
                           jnp.full((pad,), N_TRASH, jnp.int32)])

    degp = _deg_pass(dst)
    degp3 = jnp.reshape(degp, (NC, N_PAD, 1))

    y1 = _tc1(x, W1, degp3)
    s1 = _agg64(y1, src, dst)
    y2 = _tc2(y1, s1, degp3, b1, W2)
    s2 = _agg32(y2, src, dst)
    out = _tc3(y2, s2, degp3, b2, Wc, bc)
    return out[:, 0]
